# baseline (speedup 1.0000x reference)
import functools

import jax
import jax.numpy as jnp
from jax.experimental import pallas as pl
from jax.experimental.pallas import tpu as pltpu

K = 20
LANES = 128


def _edge_kernel(x_blk_ref, x_all_ref, out_ref, *, n, d, k):
    x = x_blk_ref[0]
    xa = x_all_ref[0]
    block = x.shape[0]
    nc = n // LANES

    inner = jnp.dot(x, xa.T, preferred_element_type=jnp.float32)
    xsq = jnp.sum(x * x, axis=1, keepdims=True)
    xasq = jnp.sum(xa * xa, axis=1, keepdims=True).T
    dist = xsq + (-2.0 * inner) + xasq

    iota = jax.lax.broadcasted_iota(
        jnp.int32, (block, n), 1).astype(jnp.float32)
    nf = jnp.float32(n)
    inf = jnp.float32(jnp.inf)

    def win(a, c):
        return a[:, c * LANES:(c + 1) * LANES]

    def finish_argmin(runval, runidx):
        mval = jnp.min(runval, axis=-1, keepdims=True)
        return jnp.min(jnp.where(runval == mval, runidx, nf),
                       axis=-1, keepdims=True)

    runval = win(dist, 0)
    runidx = win(iota, 0)
    for c in range(1, nc):
        dc = win(dist, c)
        cond = dc < runval
        runidx = jnp.where(cond, win(iota, c), runidx)
        runval = jnp.minimum(dc, runval)
    first = finish_argmin(runval, runidx)

    dm_w = [win(dist, c) for c in range(nc)]
    neighbors = []
    for r in range(k):
        if r == k - 1:
            oh = (iota == first).astype(jnp.float32)
            neighbors.append(
                jnp.dot(oh, xa, preferred_element_type=jnp.float32))
            break
        ohs = []
        runval = runidx = None
        for c in range(nc):
            ic = win(iota, c)
            selc = ic == first
            ohs.append(selc.astype(jnp.float32))
            dmc = jnp.where(selc, inf, dm_w[c])
            dm_w[c] = dmc
            if c == 0:
                runval, runidx = dmc, ic
            else:
                cond = dmc < runval
                runidx = jnp.where(cond, ic, runidx)
                runval = jnp.minimum(dmc, runval)
        oh = jnp.concatenate(ohs, axis=1)
        neighbors.append(jnp.dot(oh, xa, preferred_element_type=jnp.float32))
        first = finish_argmin(runval, runidx)

    for j in range(k):
        base = j * 2 * d
        out_ref[0, :, base:base + d] = x
        out_ref[0, :, base + d:base + 2 * d] = neighbors[j] - x


def kernel(inputs):
    b, n, d = inputs.shape
    block = 128
    grid = (b, n // block)

    out = pl.pallas_call(
        functools.partial(_edge_kernel, n=n, d=d, k=K),
        grid=grid,
        in_specs=[
            pl.BlockSpec((1, block, d), lambda i, j: (i, j, 0)),
            pl.BlockSpec((1, n, d), lambda i, j: (i, 0, 0)),
        ],
        out_specs=pl.BlockSpec((1, block, 2 * d * K), lambda i, j: (i, j, 0)),
        out_shape=jax.ShapeDtypeStruct((b, n, 2 * d * K), jnp.float32),
        compiler_params=pltpu.CompilerParams(
            dimension_semantics=("parallel", "parallel")),
    )(inputs, inputs)
    return out.reshape(b, n, K, 2 * d)


# speedup vs baseline: 1.2047x; 1.2047x over previous
import functools

import jax
import jax.numpy as jnp
from jax.experimental import pallas as pl
from jax.experimental.pallas import tpu as pltpu

K = 20
LANES = 128


def _edge_kernel(x_blk_ref, x_all_ref, out_ref, *, n, d, k):
    x = x_blk_ref[0]
    xa = x_all_ref[0]
    block = x.shape[0]
    nc = n // LANES

    inner = jnp.dot(x, xa.T, preferred_element_type=jnp.float32)
    xsq = jnp.sum(x * x, axis=1, keepdims=True)
    xasq = jnp.sum(xa * xa, axis=1, keepdims=True).T
    dist = xsq + (-2.0 * inner) + xasq

    iota = jax.lax.broadcasted_iota(
        jnp.int32, (block, n), 1).astype(jnp.float32)
    nf = jnp.float32(n)
    inf = jnp.float32(jnp.inf)

    def win(a, c):
        return a[:, c * LANES:(c + 1) * LANES]

    def finish_argmin(runval, runidx):
        mval = jnp.min(runval, axis=-1, keepdims=True)
        return jnp.min(jnp.where(runval == mval, runidx, nf),
                       axis=-1, keepdims=True)

    runval = win(dist, 0)
    runidx = win(iota, 0)
    for c in range(1, nc):
        dc = win(dist, c)
        cond = dc < runval
        runidx = jnp.where(cond, win(iota, c), runidx)
        runval = jnp.minimum(dc, runval)
    first = finish_argmin(runval, runidx)

    dm_w = [win(dist, c) for c in range(nc)]
    neighbors = []
    for r in range(k):
        if r == k - 1:
            oh = (iota == first).astype(jnp.float32)
            neighbors.append(
                jnp.dot(oh, xa, preferred_element_type=jnp.float32))
            break
        ohs = []
        runval = runidx = None
        for c in range(nc):
            ic = win(iota, c)
            selc = ic == first
            ohs.append(selc.astype(jnp.float32))
            dmc = jnp.where(selc, inf, dm_w[c])
            dm_w[c] = dmc
            if c == 0:
                runval, runidx = dmc, ic
            else:
                cond = dmc < runval
                runidx = jnp.where(cond, ic, runidx)
                runval = jnp.minimum(dmc, runval)
        oh = jnp.concatenate(ohs, axis=1)
        neighbors.append(jnp.dot(oh, xa, preferred_element_type=jnp.float32))
        first = finish_argmin(runval, runidx)

    for j in range(k):
        base = j * 2 * d
        out_ref[0, :, base:base + d] = x
        out_ref[0, :, base + d:base + 2 * d] = neighbors[j] - x


def kernel(inputs):
    b, n, d = inputs.shape
    block = 256
    grid = (b, n // block)

    out = pl.pallas_call(
        functools.partial(_edge_kernel, n=n, d=d, k=K),
        grid=grid,
        in_specs=[
            pl.BlockSpec((1, block, d), lambda i, j: (i, j, 0)),
            pl.BlockSpec((1, n, d), lambda i, j: (i, 0, 0)),
        ],
        out_specs=pl.BlockSpec((1, block, 2 * d * K), lambda i, j: (i, j, 0)),
        out_shape=jax.ShapeDtypeStruct((b, n, 2 * d * K), jnp.float32),
        compiler_params=pltpu.CompilerParams(
            dimension_semantics=("parallel", "parallel")),
    )(inputs, inputs)
    return out.reshape(b, n, K, 2 * d)
